# baseline (speedup 1.0000x reference)
import functools

import jax
import jax.numpy as jnp
from jax import lax
from jax.experimental import pallas as pl
from jax.experimental.pallas import tpu as pltpu
from jax.experimental.pallas import tpu_sc as plsc

N = 64
H = 512
W = 512

N_SC = 32
N_TC = N - N_SC

NUM_CORES = 2
NUM_SUBCORES = 16
NUM_WORKERS = NUM_CORES * NUM_SUBCORES

ROWS = 32
SLABS_PER_N = H // ROWS
TOTAL_SLABS = N_SC * SLABS_PER_N
SLABS_PER_WORKER = TOTAL_SLABS // NUM_WORKERS
SLOTS = 3
LANES = 16
SEGS = W // LANES
VECS = ROWS * SEGS
UNROLL = 8


def _sc_body(frames, ratio_h, dir_h, out, rv, dv, in0, in1, sin, sout):
    wid = lax.axis_index("s") * NUM_CORES + lax.axis_index("c")

    pltpu.sync_copy(ratio_h, rv)
    pltpu.sync_copy(dir_h, dv)

    one = jnp.full((LANES,), 1.0, jnp.float32)
    zero = jnp.full((LANES,), 0.0, jnp.float32)

    def slab_addr(k):
        g = wid * SLABS_PER_WORKER + k
        n = g // SLABS_PER_N
        row0 = (g % SLABS_PER_N) * ROWS
        return n, row0

    def issue_in(k):
        s = k % SLOTS
        n, row0 = slab_addr(k)
        h0 = pltpu.async_copy(frames.at[n, 0, pl.ds(row0, ROWS), :],
                              in0.at[s], sin[s])
        h1 = pltpu.async_copy(frames.at[n, 1, pl.ds(row0, ROWS), :],
                              in1.at[s], sin[s])
        return h0, h1

    n0, _ = slab_addr(0)
    r = rv[n0, :]
    d = dv[n0, :]
    av = jnp.where(d > 0, one - r, jnp.where(d < 0, r, one))
    bv = jnp.where(d > 0, r, jnp.where(d < 0, one - r, zero))

    pending_in = [None] * SLOTS
    pending_out = [None] * SLOTS
    for k in range(min(SLOTS - 1, SLABS_PER_WORKER)):
        pending_in[k % SLOTS] = issue_in(k)

    for k in range(SLABS_PER_WORKER):
        s = k % SLOTS
        for h in pending_in[s]:
            h.wait()

        @plsc.parallel_loop(0, VECS, step=1, unroll=UNROLL)
        def blend(i):
            row = i // SEGS
            c = (i % SEGS) * LANES
            x0 = in0[s, row, pl.ds(c, LANES)]
            x1 = in1[s, row, pl.ds(c, LANES)]
            in0[s, row, pl.ds(c, LANES)] = av * x0 + bv * x1

        n, row0 = slab_addr(k)
        pending_out[s] = pltpu.async_copy(
            in0.at[s], out.at[n, 0, pl.ds(row0, ROWS), :], sout[s])
        nk = k + SLOTS - 1
        if nk < SLABS_PER_WORKER:
            ns = nk % SLOTS
            if pending_out[ns] is not None:
                pending_out[ns].wait()
            pending_in[ns] = issue_in(nk)
    for s in range(SLOTS):
        if pending_out[s] is not None:
            pending_out[s].wait()


_sc_call = functools.partial(
    pl.kernel,
    mesh=plsc.VectorSubcoreMesh(core_axis_name="c", subcore_axis_name="s"),
    out_type=jax.ShapeDtypeStruct((N_SC, 1, H, W), jnp.float32),
    compiler_params=pltpu.CompilerParams(use_tc_tiling_on_sc=True),
    scratch_types=[
        pltpu.VMEM((N, LANES), jnp.float32),
        pltpu.VMEM((N, LANES), jnp.float32),
        pltpu.VMEM((SLOTS, ROWS, W), jnp.float32),
        pltpu.VMEM((SLOTS, ROWS, W), jnp.float32),
        [pltpu.SemaphoreType.DMA] * SLOTS,
        [pltpu.SemaphoreType.DMA] * SLOTS,
    ],
)(_sc_body)


def _tc_body(ratio_ref, dir_ref, frames_ref, out_ref):
    n = pl.program_id(0) + N_SC
    r = ratio_ref[n, 0]
    d = dir_ref[n, 0]
    one = jnp.float32(1.0)
    a = jnp.where(d > 0, one - r, jnp.where(d < 0, r, one))
    b = jnp.where(d > 0, r, jnp.where(d < 0, one - r, jnp.float32(0.0)))
    f0 = frames_ref[0, 0]
    f1 = frames_ref[0, 1]
    out_ref[0, 0] = a * f0 + b * f1


_tc_call = pl.pallas_call(
    _tc_body,
    grid=(N_TC,),
    in_specs=[
        pl.BlockSpec(memory_space=pltpu.SMEM),
        pl.BlockSpec(memory_space=pltpu.SMEM),
        pl.BlockSpec((1, 2, H, W), lambda n: (n + N_SC, 0, 0, 0)),
    ],
    out_specs=pl.BlockSpec((1, 1, H, W), lambda n: (n, 0, 0, 0)),
    out_shape=jax.ShapeDtypeStruct((N_TC, 1, H, W), jnp.float32),
)


def kernel(exist_frames, ratio, direction):
    ratio_b = jnp.broadcast_to(ratio.reshape(N, 1), (N, LANES))
    dir_b = jnp.broadcast_to(direction.reshape(N, 1), (N, LANES))
    out_sc = _sc_call(exist_frames, ratio_b, dir_b)
    out_tc = _tc_call(ratio, direction, exist_frames)
    return jnp.concatenate([out_sc, out_tc], axis=0)


# speedup vs baseline: 1.3998x; 1.3998x over previous
import functools

import jax
import jax.numpy as jnp
from jax import lax
from jax.experimental import pallas as pl
from jax.experimental.pallas import tpu as pltpu
from jax.experimental.pallas import tpu_sc as plsc

N = 64
H = 512
W = 512

NUM_CORES = 2
NUM_SUBCORES = 16
NUM_WORKERS = NUM_CORES * NUM_SUBCORES
N_PER_WORKER = N // NUM_WORKERS

ROWS = 32
SLABS_PER_N = H // ROWS
TOTAL_SLABS = N_PER_WORKER * SLABS_PER_N
SLOTS = 3
LANES = 16
SEGS = W // LANES
VECS = ROWS * SEGS
UNROLL = 8


def _sc_body(frames, ratio_h, dir_h, out, rv, dv, in0, in1, sin, sout):
    wid = lax.axis_index("s") * NUM_CORES + lax.axis_index("c")

    pltpu.sync_copy(ratio_h, rv)
    pltpu.sync_copy(dir_h, dv)

    one = jnp.full((LANES,), 1.0, jnp.float32)
    zero = jnp.full((LANES,), 0.0, jnp.float32)
    ws = []
    for j in range(N_PER_WORKER):
        n = wid * N_PER_WORKER + j
        r = rv[n, :]
        d = dv[n, :]
        ws.append((jnp.where(d > 0, one - r, jnp.where(d < 0, r, one)),
                   jnp.where(d > 0, r, jnp.where(d < 0, one - r, zero))))

    def slab_addr(g):
        n = wid * N_PER_WORKER + g // SLABS_PER_N
        row0 = (g % SLABS_PER_N) * ROWS
        return n, row0

    def issue_in(g, s):
        n, row0 = slab_addr(g)
        pltpu.async_copy(frames.at[n, 0, pl.ds(row0, ROWS), :],
                         in0.at[s], sin.at[s])
        pltpu.async_copy(frames.at[n, 1, pl.ds(row0, ROWS), :],
                         in1.at[s], sin.at[s])

    issue_in(0, 0)
    issue_in(1, 1)

    def step(g, carry):
        s = lax.rem(g, SLOTS)
        n, row0 = slab_addr(g)
        pltpu.make_async_copy(frames.at[n, 0, pl.ds(row0, ROWS), :],
                              in0.at[s], sin.at[s]).wait()
        pltpu.make_async_copy(frames.at[n, 1, pl.ds(row0, ROWS), :],
                              in1.at[s], sin.at[s]).wait()

        sel = g >= SLABS_PER_N
        av = jnp.where(sel, ws[1][0], ws[0][0])
        bv = jnp.where(sel, ws[1][1], ws[0][1])

        @plsc.parallel_loop(0, VECS, step=1, unroll=UNROLL)
        def blend(i):
            row = i // SEGS
            c = (i % SEGS) * LANES
            x0 = in0[s, row, pl.ds(c, LANES)]
            x1 = in1[s, row, pl.ds(c, LANES)]
            in0[s, row, pl.ds(c, LANES)] = av * x0 + bv * x1

        pltpu.async_copy(in0.at[s], out.at[n, 0, pl.ds(row0, ROWS), :],
                         sout.at[s])

        s2 = lax.rem(g + 2, SLOTS)
        n2, row2 = slab_addr(lax.min(g + 2, TOTAL_SLABS - 1))

        @pl.when(jnp.logical_and(g >= 1, g + 2 < TOTAL_SLABS))
        def _():
            pltpu.make_async_copy(in0.at[s2],
                                  out.at[n2, 0, pl.ds(row2, ROWS), :],
                                  sout.at[s2]).wait()

        @pl.when(g + 2 < TOTAL_SLABS)
        def _():
            pltpu.async_copy(frames.at[n2, 0, pl.ds(row2, ROWS), :],
                             in0.at[s2], sin.at[s2])
            pltpu.async_copy(frames.at[n2, 1, pl.ds(row2, ROWS), :],
                             in1.at[s2], sin.at[s2])
        return carry

    lax.fori_loop(0, TOTAL_SLABS, step, 0)

    n_last, _ = slab_addr(0)
    for s in range(SLOTS):
        pltpu.make_async_copy(in0.at[s],
                              out.at[n_last, 0, pl.ds(0, ROWS), :],
                              sout.at[s]).wait()


_sc_call = functools.partial(
    pl.kernel,
    mesh=plsc.VectorSubcoreMesh(core_axis_name="c", subcore_axis_name="s"),
    out_type=jax.ShapeDtypeStruct((N, 1, H, W), jnp.float32),
    compiler_params=pltpu.CompilerParams(use_tc_tiling_on_sc=True),
    scratch_types=[
        pltpu.VMEM((N, LANES), jnp.float32),
        pltpu.VMEM((N, LANES), jnp.float32),
        pltpu.VMEM((SLOTS, ROWS, W), jnp.float32),
        pltpu.VMEM((SLOTS, ROWS, W), jnp.float32),
        pltpu.SemaphoreType.DMA((SLOTS,)),
        pltpu.SemaphoreType.DMA((SLOTS,)),
    ],
)(_sc_body)


def kernel(exist_frames, ratio, direction):
    ratio_b = jnp.broadcast_to(ratio.reshape(N, 1), (N, LANES))
    dir_b = jnp.broadcast_to(direction.reshape(N, 1), (N, LANES))
    return _sc_call(exist_frames, ratio_b, dir_b)


# speedup vs baseline: 1.4235x; 1.0169x over previous
import functools

import jax
import jax.numpy as jnp
from jax import lax
from jax.experimental import pallas as pl
from jax.experimental.pallas import tpu as pltpu
from jax.experimental.pallas import tpu_sc as plsc

N = 64
H = 512
W = 512

NUM_CORES = 2
NUM_SUBCORES = 16
NUM_WORKERS = NUM_CORES * NUM_SUBCORES
N_PER_WORKER = N // NUM_WORKERS

ROWS = 32
SLABS_PER_N = H // ROWS
TOTAL_SLABS = N_PER_WORKER * SLABS_PER_N
SLOTS = 3
LANES = 16
SEGS = W // LANES
VECS = ROWS * SEGS
UNROLL = 8


def _sc_body(frames, rd_h, out, rv, in0, in1, sin, sout):
    wid = lax.axis_index("s") * NUM_CORES + lax.axis_index("c")

    pltpu.sync_copy(rd_h, rv)

    one = jnp.full((LANES,), 1.0, jnp.float32)
    zero = jnp.full((LANES,), 0.0, jnp.float32)
    ws = []
    for j in range(N_PER_WORKER):
        n = wid * N_PER_WORKER + j
        r = rv[n, :]
        d = rv[N + n, :]
        ws.append((jnp.where(d > 0, one - r, jnp.where(d < 0, r, one)),
                   jnp.where(d > 0, r, jnp.where(d < 0, one - r, zero))))

    def slab_addr(g):
        n = wid * N_PER_WORKER + g // SLABS_PER_N
        row0 = (g % SLABS_PER_N) * ROWS
        return n, row0

    def issue_in(g, s):
        n, row0 = slab_addr(g)
        pltpu.async_copy(frames.at[n, 0, pl.ds(row0, ROWS), :],
                         in0.at[s], sin.at[s])
        pltpu.async_copy(frames.at[n, 1, pl.ds(row0, ROWS), :],
                         in1.at[s], sin.at[s])

    issue_in(0, 0)
    issue_in(1, 1)

    def step(g, carry):
        s = lax.rem(g, SLOTS)
        n, row0 = slab_addr(g)
        pltpu.make_async_copy(frames.at[n, 0, pl.ds(row0, ROWS), :],
                              in0.at[s], sin.at[s]).wait()
        pltpu.make_async_copy(frames.at[n, 1, pl.ds(row0, ROWS), :],
                              in1.at[s], sin.at[s]).wait()

        sel = g >= SLABS_PER_N
        av = jnp.where(sel, ws[1][0], ws[0][0])
        bv = jnp.where(sel, ws[1][1], ws[0][1])

        @plsc.parallel_loop(0, VECS, step=1, unroll=UNROLL)
        def blend(i):
            row = i // SEGS
            c = (i % SEGS) * LANES
            x0 = in0[s, row, pl.ds(c, LANES)]
            x1 = in1[s, row, pl.ds(c, LANES)]
            in0[s, row, pl.ds(c, LANES)] = av * x0 + bv * x1

        pltpu.async_copy(in0.at[s], out.at[n, 0, pl.ds(row0, ROWS), :],
                         sout.at[s])

        s2 = lax.rem(g + 2, SLOTS)
        n2, row2 = slab_addr(lax.min(g + 2, TOTAL_SLABS - 1))

        @pl.when(jnp.logical_and(g >= 1, g + 2 < TOTAL_SLABS))
        def _():
            pltpu.make_async_copy(in0.at[s2],
                                  out.at[n2, 0, pl.ds(row2, ROWS), :],
                                  sout.at[s2]).wait()

        @pl.when(g + 2 < TOTAL_SLABS)
        def _():
            pltpu.async_copy(frames.at[n2, 0, pl.ds(row2, ROWS), :],
                             in0.at[s2], sin.at[s2])
            pltpu.async_copy(frames.at[n2, 1, pl.ds(row2, ROWS), :],
                             in1.at[s2], sin.at[s2])
        return carry

    lax.fori_loop(0, TOTAL_SLABS, step, 0)

    n_last, _ = slab_addr(0)
    for s in range(SLOTS):
        pltpu.make_async_copy(in0.at[s],
                              out.at[n_last, 0, pl.ds(0, ROWS), :],
                              sout.at[s]).wait()


_sc_call = functools.partial(
    pl.kernel,
    mesh=plsc.VectorSubcoreMesh(core_axis_name="c", subcore_axis_name="s"),
    out_type=jax.ShapeDtypeStruct((N, 1, H, W), jnp.float32),
    compiler_params=pltpu.CompilerParams(use_tc_tiling_on_sc=True),
    scratch_types=[
        pltpu.VMEM((2 * N, LANES), jnp.float32),
        pltpu.VMEM((SLOTS, ROWS, W), jnp.float32),
        pltpu.VMEM((SLOTS, ROWS, W), jnp.float32),
        pltpu.SemaphoreType.DMA((SLOTS,)),
        pltpu.SemaphoreType.DMA((SLOTS,)),
    ],
)(_sc_body)


def kernel(exist_frames, ratio, direction):
    rd = jnp.concatenate([ratio, direction], axis=0)
    rd_b = jnp.broadcast_to(rd, (2 * N, LANES))
    return _sc_call(exist_frames, rd_b)
